# Initial kernel scaffold; baseline (speedup 1.0000x reference)
import jax
import jax.numpy as jnp
from jax.experimental import pallas as pl


def kernel(x, edge_index, W1_l, b1_l, W1_r, W2_l, b2_l, W2_r, W_cls, b_cls):
    raise NotImplementedError("write your pallas kernel here")


# speedup vs baseline: 2.9486x; 2.9486x over previous
import functools
import jax
import jax.numpy as jnp
from jax import lax
from jax.experimental import pallas as pl
from jax.experimental.pallas import tpu as pltpu
from jax.experimental.pallas import tpu_sc as plsc

N_NODES = 10000
D = 128
NC = 2
NS = 16
NW = NC * NS
CH = 128
SPC = 16
NPAD = 10112
RPT = NPAD // NS
CROW = 80


def _sc_agg_body(with_cnt, nchunk, *refs):
    if with_cnt:
        (x_hbm, src_hbm, dst_hbm, z_hbm, out_hbm, cnt_hbm,
         src_v, dst_v, rows_v, cntp_v, acc_sh, sem) = refs
    else:
        (x_hbm, src_hbm, dst_hbm, z_hbm, out_hbm,
         src_v, dst_v, rows_v, acc_sh, sem) = refs
    c = lax.axis_index("c")
    s = lax.axis_index("s")
    wid = s * NC + c

    pltpu.sync_copy(z_hbm, rows_v)
    if with_cnt:
        def zcnt(i, _):
            cntp_v[pl.ds(i * 16, 16)] = jnp.zeros((16,), jnp.float32)
            return 0
        lax.fori_loop(0, CROW * CH // 16, zcnt, 0)
    nfull = RPT // CH
    rem = RPT % CH
    for k in range(nfull):
        pltpu.sync_copy(rows_v, acc_sh.at[pl.ds(s * RPT + k * CH, CH)])
    if rem:
        pltpu.sync_copy(rows_v.at[pl.ds(0, rem)],
                        acc_sh.at[pl.ds(s * RPT + nfull * CH, rem)])

    plsc.subcore_barrier()

    ones = jnp.ones((16,), jnp.float32)

    def stage(st, _):
        pltpu.sync_copy(src_hbm.at[wid, pl.ds(st * SPC, SPC)], src_v)
        pltpu.sync_copy(dst_hbm.at[wid, pl.ds(st * SPC, SPC)], dst_v)

        def chunk(j, _):
            cp = pltpu.async_copy(x_hbm.at[src_v.at[j]], rows_v, sem)
            if with_cnt:
                for k in range(CH // 16):
                    iv = dst_v[j, pl.ds(k * 16, 16)]
                    plsc.addupdate_scatter(cntp_v, [iv], ones)
            cp.wait()
            pltpu.sync_copy(rows_v, acc_sh.at[dst_v.at[j]], add=True)
            return 0
        lax.fori_loop(0, SPC, chunk, 0)
        return 0
    lax.fori_loop(0, nchunk // SPC, stage, 0)

    plsc.subcore_barrier()

    for k in range(nfull + (1 if rem else 0)):
        w = CH if k < nfull else rem
        base = s * RPT + k * CH
        pltpu.sync_copy(acc_sh.at[pl.ds(base, w)], rows_v.at[pl.ds(0, w)])
        pltpu.sync_copy(rows_v.at[pl.ds(0, w)],
                        out_hbm.at[c, pl.ds(base, w)])
    if with_cnt:
        pltpu.sync_copy(cntp_v,
                        cnt_hbm.at[pl.ds(wid * CROW * CH, CROW * CH)])


def _make_sc_agg(with_cnt, nchunk):
    mesh = plsc.VectorSubcoreMesh(core_axis_name="c", subcore_axis_name="s")
    out_type = [jax.ShapeDtypeStruct((NC, NPAD, D), jnp.float32)]
    scratch = [
        pltpu.VMEM((SPC, CH), jnp.int32),
        pltpu.VMEM((SPC, CH), jnp.int32),
        pltpu.VMEM((CH, D), jnp.float32),
    ]
    if with_cnt:
        out_type.append(
            jax.ShapeDtypeStruct((NW * CROW * CH,), jnp.float32))
        scratch.append(pltpu.VMEM((CROW * CH,), jnp.float32))
    scratch.append(pltpu.VMEM_SHARED((NPAD, D), jnp.float32))
    scratch.append(pltpu.SemaphoreType.DMA)
    params = pltpu.CompilerParams(needs_layout_passes=False)
    return pl.kernel(
        functools.partial(_sc_agg_body, with_cnt, nchunk),
        out_type=out_type,
        mesh=mesh,
        compiler_params=params,
        scratch_types=scratch,
    )


def _cnt_inv_tc(cnt):
    def body(c_ref, o_ref):
        tot = jnp.sum(c_ref[...], axis=0)
        o_ref[...] = 1.0 / jnp.maximum(tot, 1.0)

    return pl.pallas_call(
        body,
        grid=(1,),
        in_specs=[pl.BlockSpec((NW, CROW, CH), lambda i: (0, 0, 0))],
        out_specs=pl.BlockSpec((CROW, CH), lambda i: (0, 0)),
        out_shape=jax.ShapeDtypeStruct((CROW, CH), jnp.float32),
    )(cnt)


def _layer1_tc(acc, inv, xin, W_l, b_l, W_r):
    R = 400
    grid = (N_NODES // R,)

    def body(acc_ref, inv_ref, x_ref, wl_ref, bl_ref, wr_ref, o_ref):
        agg = (acc_ref[0] + acc_ref[1]) * inv_ref[...]
        h = lax.dot_general(agg, wl_ref[...], (((1,), (1,)), ((), ())),
                            preferred_element_type=jnp.float32)
        h = h + lax.dot_general(x_ref[...], wr_ref[...],
                                (((1,), (1,)), ((), ())),
                                preferred_element_type=jnp.float32)
        h = h + bl_ref[...]
        o_ref[...] = jnp.maximum(h, 0.0)

    return pl.pallas_call(
        body,
        grid=grid,
        in_specs=[
            pl.BlockSpec((NC, R, D), lambda i: (0, i, 0)),
            pl.BlockSpec((R, 1), lambda i: (i, 0)),
            pl.BlockSpec((R, D), lambda i: (i, 0)),
            pl.BlockSpec((D, D), lambda i: (0, 0)),
            pl.BlockSpec((1, D), lambda i: (0, 0)),
            pl.BlockSpec((D, D), lambda i: (0, 0)),
        ],
        out_specs=pl.BlockSpec((R, D), lambda i: (i, 0)),
        out_shape=jax.ShapeDtypeStruct((N_NODES, D), jnp.float32),
    )(acc, inv, xin, W_l, b_l, W_r)


def _layer2_tc(acc, inv, hin, W_l, b_l, W_r, W_cls, b_cls):
    R = 400
    grid = (N_NODES // R,)

    def body(acc_ref, inv_ref, h_ref, wl_ref, bl_ref, wr_ref, wc_ref,
             bc_ref, o_ref):
        agg = (acc_ref[0] + acc_ref[1]) * inv_ref[...]
        h = lax.dot_general(agg, wl_ref[...], (((1,), (1,)), ((), ())),
                            preferred_element_type=jnp.float32)
        h = h + lax.dot_general(h_ref[...], wr_ref[...],
                                (((1,), (1,)), ((), ())),
                                preferred_element_type=jnp.float32)
        h = h + bl_ref[...]
        h = jnp.maximum(h, 0.0)
        res = jnp.sum(h * wc_ref[...], axis=1) + bc_ref[0, 0]
        o_ref[...] = res.reshape(R, 1)

    return pl.pallas_call(
        body,
        grid=grid,
        in_specs=[
            pl.BlockSpec((NC, R, D), lambda i: (0, i, 0)),
            pl.BlockSpec((R, 1), lambda i: (i, 0)),
            pl.BlockSpec((R, D), lambda i: (i, 0)),
            pl.BlockSpec((D, D), lambda i: (0, 0)),
            pl.BlockSpec((1, D), lambda i: (0, 0)),
            pl.BlockSpec((D, D), lambda i: (0, 0)),
            pl.BlockSpec((1, D), lambda i: (0, 0)),
            pl.BlockSpec((1, 1), lambda i: (0, 0)),
        ],
        out_specs=pl.BlockSpec((R, 1), lambda i: (i, 0)),
        out_shape=jax.ShapeDtypeStruct((N_NODES, 1), jnp.float32),
    )(acc, inv, hin, W_l, b_l, W_r, W_cls, b_cls)


@jax.jit
def kernel(x, edge_index, W1_l, b1_l, W1_r, W2_l, b2_l, W2_r, W_cls, b_cls):
    n_edges = edge_index.shape[1]
    ept = -(-n_edges // NW)
    ept = -(-ept // (SPC * CH)) * (SPC * CH)
    nchunk = ept // CH
    epad = ept * NW

    src = edge_index[0].astype(jnp.int32)
    dst = edge_index[1].astype(jnp.int32)
    src_p = jnp.concatenate(
        [src, jnp.zeros((epad - n_edges,), jnp.int32)]).reshape(NW, nchunk, CH)
    dst_p = jnp.concatenate(
        [dst, jnp.full((epad - n_edges,), N_NODES, jnp.int32)]
    ).reshape(NW, nchunk, CH)

    zrows = jnp.zeros((CH, D), jnp.float32)
    agg1, cnt = _make_sc_agg(True, nchunk)(x, src_p, dst_p, zrows)
    inv = _cnt_inv_tc(cnt.reshape(NW, CROW, CH))
    inv = inv.reshape(CROW * CH, 1)[:N_NODES]
    h1 = _layer1_tc(agg1, inv, x, W1_l, b1_l.reshape(1, D), W1_r)
    (agg2,) = _make_sc_agg(False, nchunk)(h1, src_p, dst_p, zrows)
    out = _layer2_tc(agg2, inv, h1, W2_l, b2_l.reshape(1, D), W2_r,
                     W_cls, b_cls.reshape(1, 1))
    return out.reshape(N_NODES)


# speedup vs baseline: 3.1364x; 1.0637x over previous
import functools
import jax
import jax.numpy as jnp
from jax import lax
from jax.experimental import pallas as pl
from jax.experimental.pallas import tpu as pltpu
from jax.experimental.pallas import tpu_sc as plsc

N_NODES = 10000
D = 128
NC = 2
NS = 16
NW = NC * NS
CH = 128
SPC = 8
NPAD = 10112
RPT = NPAD // NS
CROW = 80
CSPC = 16


def _sc_agg_body(nchunk, x_hbm, src_hbm, dst_hbm, z_hbm, out_hbm,
                 src_v, dst_v, rows0_v, rows1_v,
                 acc_sh, sg0, sg1, ss0, ss1):
    c = lax.axis_index("c")
    s = lax.axis_index("s")
    wid = s * NC + c

    pltpu.sync_copy(z_hbm, rows0_v)
    nfull = RPT // CH
    rem = RPT % CH
    for k in range(nfull):
        pltpu.sync_copy(rows0_v, acc_sh.at[pl.ds(s * RPT + k * CH, CH)])
    if rem:
        pltpu.sync_copy(rows0_v.at[pl.ds(0, rem)],
                        acc_sh.at[pl.ds(s * RPT + nfull * CH, rem)])

    plsc.subcore_barrier()

    rows = (rows0_v, rows1_v)
    sg = (sg0, sg1)
    ss = (ss0, ss1)

    def stage(st, _):
        pltpu.sync_copy(src_hbm.at[wid, pl.ds(st * SPC, SPC)], src_v)
        pltpu.sync_copy(dst_hbm.at[wid, pl.ds(st * SPC, SPC)], dst_v)
        pltpu.async_copy(x_hbm.at[src_v.at[0]], rows[0], sg[0])
        for j in range(SPC):
            b = j & 1
            if j + 1 < SPC:
                if j >= 1:
                    pltpu.make_async_copy(
                        rows[1 - b], acc_sh.at[dst_v.at[j - 1]],
                        ss[1 - b]).wait()
                pltpu.async_copy(x_hbm.at[src_v.at[j + 1]],
                                 rows[1 - b], sg[1 - b])
            pltpu.make_async_copy(x_hbm.at[src_v.at[j]], rows[b],
                                  sg[b]).wait()
            pltpu.async_copy(rows[b], acc_sh.at[dst_v.at[j]], ss[b],
                             add=True)
        for b in range(2):
            pltpu.make_async_copy(
                rows[b], acc_sh.at[dst_v.at[SPC - 2 + b]], ss[b]).wait()
        return 0
    lax.fori_loop(0, nchunk // SPC, stage, 0)

    plsc.subcore_barrier()

    for k in range(nfull + (1 if rem else 0)):
        w = CH if k < nfull else rem
        base = s * RPT + k * CH
        pltpu.sync_copy(acc_sh.at[pl.ds(base, w)], rows0_v.at[pl.ds(0, w)])
        pltpu.sync_copy(rows0_v.at[pl.ds(0, w)],
                        out_hbm.at[c, pl.ds(base, w)])


def _make_sc_agg(nchunk):
    mesh = plsc.VectorSubcoreMesh(core_axis_name="c", subcore_axis_name="s")
    return pl.kernel(
        functools.partial(_sc_agg_body, nchunk),
        out_type=[jax.ShapeDtypeStruct((NC, NPAD, D), jnp.float32)],
        mesh=mesh,
        compiler_params=pltpu.CompilerParams(needs_layout_passes=False),
        scratch_types=[
            pltpu.VMEM((SPC, CH), jnp.int32),
            pltpu.VMEM((SPC, CH), jnp.int32),
            pltpu.VMEM((CH, D), jnp.float32),
            pltpu.VMEM((CH, D), jnp.float32),
            pltpu.VMEM_SHARED((NPAD, D), jnp.float32),
            pltpu.SemaphoreType.DMA,
            pltpu.SemaphoreType.DMA,
            pltpu.SemaphoreType.DMA,
            pltpu.SemaphoreType.DMA,
        ],
    )


def _sc_cnt_body(nchunk, dst_hbm, cnt_hbm, dst_v, cntp_v, sem):
    c = lax.axis_index("c")
    s = lax.axis_index("s")
    wid = s * NC + c

    def zcnt(i, _):
        cntp_v[pl.ds(i * 16, 16)] = jnp.zeros((16,), jnp.float32)
        return 0
    lax.fori_loop(0, CROW * CH // 16, zcnt, 0)

    ones = jnp.ones((16,), jnp.float32)

    def stage(st, _):
        pltpu.sync_copy(dst_hbm.at[wid, pl.ds(st * CSPC, CSPC)], dst_v)

        def chunk(j, _):
            for k in range(CH // 16):
                iv = dst_v[j, pl.ds(k * 16, 16)]
                plsc.addupdate_scatter(cntp_v, [iv], ones)
            return 0
        lax.fori_loop(0, CSPC, chunk, 0)
        return 0
    lax.fori_loop(0, nchunk // CSPC, stage, 0)

    pltpu.sync_copy(cntp_v, cnt_hbm.at[pl.ds(wid * CROW * CH, CROW * CH)])


def _make_sc_cnt(nchunk):
    mesh = plsc.VectorSubcoreMesh(core_axis_name="c", subcore_axis_name="s")
    return pl.kernel(
        functools.partial(_sc_cnt_body, nchunk),
        out_type=[jax.ShapeDtypeStruct((NW * CROW * CH,), jnp.float32)],
        mesh=mesh,
        compiler_params=pltpu.CompilerParams(needs_layout_passes=False),
        scratch_types=[
            pltpu.VMEM((CSPC, CH), jnp.int32),
            pltpu.VMEM((CROW * CH,), jnp.float32),
            pltpu.SemaphoreType.DMA,
        ],
    )


def _cnt_inv_tc(cnt):
    def body(c_ref, o_ref):
        tot = jnp.sum(c_ref[...], axis=0)
        o_ref[...] = 1.0 / jnp.maximum(tot, 1.0)

    return pl.pallas_call(
        body,
        grid=(1,),
        in_specs=[pl.BlockSpec((NW, CROW, CH), lambda i: (0, 0, 0))],
        out_specs=pl.BlockSpec((CROW, CH), lambda i: (0, 0)),
        out_shape=jax.ShapeDtypeStruct((CROW, CH), jnp.float32),
    )(cnt)


def _layer1_tc(acc, inv, xin, W_l, b_l, W_r):
    R = 400
    grid = (N_NODES // R,)

    def body(acc_ref, inv_ref, x_ref, wl_ref, bl_ref, wr_ref, o_ref):
        agg = (acc_ref[0] + acc_ref[1]) * inv_ref[...]
        h = lax.dot_general(agg, wl_ref[...], (((1,), (1,)), ((), ())),
                            preferred_element_type=jnp.float32)
        h = h + lax.dot_general(x_ref[...], wr_ref[...],
                                (((1,), (1,)), ((), ())),
                                preferred_element_type=jnp.float32)
        h = h + bl_ref[...]
        o_ref[...] = jnp.maximum(h, 0.0)

    return pl.pallas_call(
        body,
        grid=grid,
        in_specs=[
            pl.BlockSpec((NC, R, D), lambda i: (0, i, 0)),
            pl.BlockSpec((R, 1), lambda i: (i, 0)),
            pl.BlockSpec((R, D), lambda i: (i, 0)),
            pl.BlockSpec((D, D), lambda i: (0, 0)),
            pl.BlockSpec((1, D), lambda i: (0, 0)),
            pl.BlockSpec((D, D), lambda i: (0, 0)),
        ],
        out_specs=pl.BlockSpec((R, D), lambda i: (i, 0)),
        out_shape=jax.ShapeDtypeStruct((N_NODES, D), jnp.float32),
    )(acc, inv, xin, W_l, b_l, W_r)


def _layer2_tc(acc, inv, hin, W_l, b_l, W_r, W_cls, b_cls):
    R = 400
    grid = (N_NODES // R,)

    def body(acc_ref, inv_ref, h_ref, wl_ref, bl_ref, wr_ref, wc_ref,
             bc_ref, o_ref):
        agg = (acc_ref[0] + acc_ref[1]) * inv_ref[...]
        h = lax.dot_general(agg, wl_ref[...], (((1,), (1,)), ((), ())),
                            preferred_element_type=jnp.float32)
        h = h + lax.dot_general(h_ref[...], wr_ref[...],
                                (((1,), (1,)), ((), ())),
                                preferred_element_type=jnp.float32)
        h = h + bl_ref[...]
        h = jnp.maximum(h, 0.0)
        res = jnp.sum(h * wc_ref[...], axis=1) + bc_ref[0, 0]
        o_ref[...] = res.reshape(R, 1)

    return pl.pallas_call(
        body,
        grid=grid,
        in_specs=[
            pl.BlockSpec((NC, R, D), lambda i: (0, i, 0)),
            pl.BlockSpec((R, 1), lambda i: (i, 0)),
            pl.BlockSpec((R, D), lambda i: (i, 0)),
            pl.BlockSpec((D, D), lambda i: (0, 0)),
            pl.BlockSpec((1, D), lambda i: (0, 0)),
            pl.BlockSpec((D, D), lambda i: (0, 0)),
            pl.BlockSpec((1, D), lambda i: (0, 0)),
            pl.BlockSpec((1, 1), lambda i: (0, 0)),
        ],
        out_specs=pl.BlockSpec((R, 1), lambda i: (i, 0)),
        out_shape=jax.ShapeDtypeStruct((N_NODES, 1), jnp.float32),
    )(acc, inv, hin, W_l, b_l, W_r, W_cls, b_cls)


@jax.jit
def kernel(x, edge_index, W1_l, b1_l, W1_r, W2_l, b2_l, W2_r, W_cls, b_cls):
    n_edges = edge_index.shape[1]
    ept = -(-n_edges // NW)
    lcm = max(SPC, CSPC) * CH
    ept = -(-ept // lcm) * lcm
    nchunk = ept // CH
    epad = ept * NW

    src = edge_index[0].astype(jnp.int32)
    dst = edge_index[1].astype(jnp.int32)
    src_p = jnp.concatenate(
        [src, jnp.zeros((epad - n_edges,), jnp.int32)]).reshape(NW, nchunk, CH)
    dst_p = jnp.concatenate(
        [dst, jnp.full((epad - n_edges,), N_NODES, jnp.int32)]
    ).reshape(NW, nchunk, CH)
    zrows = jnp.zeros((CH, D), jnp.float32)

    (cnt,) = _make_sc_cnt(nchunk)(dst_p)
    inv = _cnt_inv_tc(cnt.reshape(NW, CROW, CH))
    inv = inv.reshape(CROW * CH, 1)[:N_NODES]
    (agg1,) = _make_sc_agg(nchunk)(x, src_p, dst_p, zrows)
    h1 = _layer1_tc(agg1, inv, x, W1_l, b1_l.reshape(1, D), W1_r)
    (agg2,) = _make_sc_agg(nchunk)(h1, src_p, dst_p, zrows)
    out = _layer2_tc(agg2, inv, h1, W2_l, b2_l.reshape(1, D), W2_r,
                     W_cls, b_cls.reshape(1, 1))
    return out.reshape(N_NODES)
